# baseline (speedup 1.0000x reference)
import functools

import jax
import jax.numpy as jnp
from jax import lax
from jax.experimental import pallas as pl
from jax.experimental.pallas import tpu as pltpu
from jax.experimental.pallas import tpu_sc as plsc

_BATCH = 4
_SEQ = 4096
_DIM = 1024
_NC = 2
_NS = 16
_SC_ROWS = _SEQ // _NC

_SP_ROWS = 968
_SP_CHUNKS = (248, 240, 240, 240)
_SP_SLOT = 248
_SP_NBUF = 3

_T_ROWS = (_SC_ROWS - _SP_ROWS) // (_NS - 1)
_T_CHUNKS = (24, 24, 24)
_T_SLOT = 24
_T_NBUF = 3


@functools.partial(
    pl.kernel,
    mesh=plsc.VectorSubcoreMesh(core_axis_name="c", subcore_axis_name="s"),
    out_type=jax.ShapeDtypeStruct((_BATCH, _SEQ, _DIM), jnp.float32),
    scratch_types=[
        pltpu.VMEM((_T_NBUF, _T_SLOT, _DIM), jnp.float32),
        pltpu.VMEM_SHARED((_SP_NBUF, _SP_SLOT, _DIM), jnp.float32),
        pltpu.SemaphoreType.DMA,
        pltpu.SemaphoreType.DMA,
        pltpu.SemaphoreType.DMA,
        pltpu.SemaphoreType.DMA,
        pltpu.SemaphoreType.DMA,
        pltpu.SemaphoreType.DMA,
        pltpu.SemaphoreType.DMA,
        pltpu.SemaphoreType.DMA,
    ],
)
def _broadcast_rows(w_hbm, out_hbm, ring, sring, rsem, w0, w1, w2, srsem, sw0, sw1, sw2):
    cid = lax.axis_index("c")
    sid = lax.axis_index("s")
    sc_base = cid * _SC_ROWS
    wsems = (w0, w1, w2)
    swsems = (sw0, sw1, sw2)

    def pipeline(base, chunks, nbuf, buf, rd_sem, wr_sems):
        nchunk = len(chunks)
        offs = [base + sum(chunks[:i]) for i in range(nchunk)]

        def rows(i):
            return pl.ds(offs[i], chunks[i])

        def slot_buf(i):
            return buf.at[i % nbuf, pl.ds(0, chunks[i])]

        reads = [pltpu.make_async_copy(w_hbm.at[rows(0)], slot_buf(0), rd_sem)]
        reads[0].start()
        writes = []
        for i in range(nchunk):
            nxt = i + 1
            if nxt < nchunk:
                if nxt >= nbuf:
                    for cp in writes[nxt - nbuf]:
                        cp.wait()
                cp = pltpu.make_async_copy(w_hbm.at[rows(nxt)], slot_buf(nxt), rd_sem)
                cp.start()
                reads.append(cp)
            reads[i].wait()
            cps = [
                pltpu.make_async_copy(slot_buf(i), out_hbm.at[b].at[rows(i)], wr_sems[i % nbuf])
                for b in range(_BATCH)
            ]
            for cp in cps:
                cp.start()
            writes.append(cps)
        for i in range(max(0, nchunk - nbuf), nchunk):
            for cp in writes[i]:
                cp.wait()

    @pl.when(sid == 0)
    def _spmem_path():
        pipeline(sc_base, _SP_CHUNKS, _SP_NBUF, sring, srsem, swsems)

    @pl.when(sid != 0)
    def _tile_path():
        base = sc_base + _SP_ROWS + (sid - 1) * _T_ROWS
        pipeline(base, _T_CHUNKS, _T_NBUF, ring, rsem, wsems)


def kernel(input, weights):
    del input
    return _broadcast_rows(weights)


# speedup vs baseline: 1.0004x; 1.0004x over previous
import functools

import jax
import jax.numpy as jnp
from jax import lax
from jax.experimental import pallas as pl
from jax.experimental.pallas import tpu as pltpu
from jax.experimental.pallas import tpu_sc as plsc

_BATCH = 4
_SEQ = 4096
_DIM = 1024
_NC = 2
_NS = 16
_NW = _NC * _NS
_ROWS_PER_W = _SEQ // _NW
_CHUNKS = (8, 40, 40, 40)
_SLOT = 40
_NBUF = 3


@functools.partial(
    pl.kernel,
    mesh=plsc.VectorSubcoreMesh(core_axis_name="c", subcore_axis_name="s"),
    out_type=jax.ShapeDtypeStruct((_BATCH, _SEQ, _DIM), jnp.float32),
    scratch_types=[
        pltpu.VMEM((_NBUF, _SLOT, _DIM), jnp.float32),
        pltpu.SemaphoreType.DMA,
        pltpu.SemaphoreType.DMA,
        pltpu.SemaphoreType.DMA,
        pltpu.SemaphoreType.DMA,
    ],
)
def _broadcast_rows(w_hbm, out_hbm, ring, rsem, wsem_0, wsem_1, wsem_2):
    wid = lax.axis_index("s") * _NC + lax.axis_index("c")
    base = wid * _ROWS_PER_W
    wsems = (wsem_0, wsem_1, wsem_2)
    nchunk = len(_CHUNKS)
    offs = [sum(_CHUNKS[:i]) for i in range(nchunk)]

    def rows(i):
        return pl.ds(base + offs[i], _CHUNKS[i])

    def slot_buf(i):
        return ring.at[i % _NBUF, pl.ds(0, _CHUNKS[i])]

    reads = [pltpu.make_async_copy(w_hbm.at[rows(0)], slot_buf(0), rsem)]
    reads[0].start()
    writes = []
    for i in range(nchunk):
        nxt = i + 1
        if nxt < nchunk:
            if nxt >= _NBUF:
                for cp in writes[nxt - _NBUF]:
                    cp.wait()
            cp = pltpu.make_async_copy(w_hbm.at[rows(nxt)], slot_buf(nxt), rsem)
            cp.start()
            reads.append(cp)
        reads[i].wait()
        cps = [
            pltpu.make_async_copy(slot_buf(i), out_hbm.at[b].at[rows(i)], wsems[i % _NBUF])
            for b in range(_BATCH)
        ]
        for cp in cps:
            cp.start()
        writes.append(cps)
    for i in range(max(0, nchunk - _NBUF), nchunk):
        for cp in writes[i]:
            cp.wait()


def kernel(input, weights):
    del input
    return _broadcast_rows(weights)


# speedup vs baseline: 1.0076x; 1.0072x over previous
import functools

import jax
import jax.numpy as jnp
from jax import lax
from jax.experimental import pallas as pl
from jax.experimental.pallas import tpu as pltpu
from jax.experimental.pallas import tpu_sc as plsc

_BATCH = 4
_SEQ = 4096
_DIM = 1024
_NC = 2
_NS = 16
_NW = _NC * _NS
_ROWS_PER_W = _SEQ // _NW
_CHUNK = 32
_NCHUNK = _ROWS_PER_W // _CHUNK
_NBUF = 3


@functools.partial(
    pl.kernel,
    mesh=plsc.VectorSubcoreMesh(core_axis_name="c", subcore_axis_name="s"),
    out_type=jax.ShapeDtypeStruct((_BATCH, _SEQ, _DIM), jnp.float32),
    scratch_types=[
        pltpu.VMEM((_NBUF, _CHUNK, _DIM), jnp.float32),
        pltpu.SemaphoreType.DMA,
        pltpu.SemaphoreType.DMA,
        pltpu.SemaphoreType.DMA,
        pltpu.SemaphoreType.DMA,
    ],
)
def _broadcast_rows(w_hbm, out_hbm, ring, rsem, wsem_0, wsem_1, wsem_2):
    wid = lax.axis_index("s") * _NC + lax.axis_index("c")
    base = wid * _ROWS_PER_W
    wsems = (wsem_0, wsem_1, wsem_2)

    def row_slice(i):
        return pl.ds(base + i * _CHUNK, _CHUNK)

    reads = []
    cp = pltpu.make_async_copy(w_hbm.at[row_slice(0)], ring.at[0], rsem)
    cp.start()
    reads.append(cp)

    writes = []
    for i in range(_NCHUNK):
        slot = i % _NBUF
        nxt = i + 1
        if nxt < _NCHUNK:
            if nxt >= _NBUF:
                for cp in writes[nxt - _NBUF]:
                    cp.wait()
            cp = pltpu.make_async_copy(
                w_hbm.at[row_slice(nxt)], ring.at[nxt % _NBUF], rsem
            )
            cp.start()
            reads.append(cp)
        reads[i].wait()
        cps = [
            pltpu.make_async_copy(ring.at[slot], out_hbm.at[b].at[row_slice(i)], wsems[slot])
            for b in range(_BATCH)
        ]
        for cp in cps:
            cp.start()
        writes.append(cps)
    for i in range(max(0, _NCHUNK - _NBUF), _NCHUNK):
        for cp in writes[i]:
            cp.wait()


def kernel(input, weights):
    del input
    return _broadcast_rows(weights)
